# baseline (speedup 1.0000x reference)
import jax
import jax.numpy as jnp
from jax.experimental import pallas as pl
from jax.experimental.pallas import tpu as pltpu

B, S, D = 2, 2048, 768
E, K = 8, 2
GS = 1024
MLP = 3072
CAP = (GS * K) // E
G = (B * S) // GS


def _router_body(x_ref, wr_ref, cols_ref, aux_ref):
    wr = wr_ref[...]
    iota_e = jax.lax.broadcasted_iota(jnp.int32, (GS, E), 1).astype(jnp.float32)
    tri =(jax.lax.broadcasted_iota(jnp.int32, (GS, GS), 1)
           < jax.lax.broadcasted_iota(jnp.int32, (GS, GS), 0)
           ).astype(jnp.float32)
    imps = []
    for g in range(G):
        xg = x_ref[g]
        logits = jax.lax.dot_general(
            xg, wr, (((1,), (0,)), ((), ())),
            preferred_element_type=jnp.float32)
        gates = jax.nn.softmax(logits, axis=-1)
        m1 = jnp.max(gates, axis=-1, keepdims=True)
        i1 = jnp.min(jnp.where(gates == m1, iota_e, float(E)), axis=-1,
                     keepdims=True)
        oh1 = (iota_e == i1).astype(jnp.float32)
        gmask = gates - oh1 * 1e30
        m2 = jnp.max(gmask, axis=-1, keepdims=True)
        i2 = jnp.min(jnp.where(gmask == m2, iota_e, float(E)), axis=-1,
                     keepdims=True)
        oh2 = (iota_e == i2).astype(jnp.float32)
        exc = jax.lax.dot_general(
            tri, jnp.concatenate([oh1, oh2], axis=1),
            (((1,), (0,)), ((), ())), preferred_element_type=jnp.float32)
        tot0 = jnp.sum(oh1, axis=0, keepdims=True)
        p0 = jnp.sum(oh1 * exc[:, :E], axis=-1, keepdims=True)
        p1 = jnp.sum(oh2 * (exc[:, E:] + tot0), axis=-1, keepdims=True)
        k0 = (p0 < CAP).astype(jnp.float32)
        k1 = (p1 < CAP).astype(jnp.float32)
        gate0 = jnp.sum(oh1 * gates, axis=-1, keepdims=True)
        gate1 = jnp.sum(oh2 * gates, axis=-1, keepdims=True)
        rows = pl.ds(g * GS, GS)
        cols_ref[rows, :] = jnp.concatenate([
            jnp.where(k0 > 0, i1 * CAP + p0, -1.0), gate0 * k0,
            jnp.where(k1 > 0, i2 * CAP + p1, -1.0), gate1 * k1,
            jnp.zeros((GS, 4), jnp.float32)], axis=1)
        imps.append(jnp.sum(gates, axis=0, keepdims=True))
    imp = jnp.concatenate(imps, axis=0)
    mean = jnp.mean(imp, axis=-1, keepdims=True)
    var = jnp.mean((imp - mean) ** 2, axis=-1, keepdims=True)
    aux = jnp.mean(var / (mean + 1e-10) ** 2)
    aux_ref[...] = jnp.full((1, 1), aux, jnp.float32)


def _moe_body(x_ref, cols_ref, w1_ref, b1_ref, w2_ref, b2_ref, out_ref,
              xe_ref, y_ref):
    e = pl.program_id(0)
    m = pl.program_id(1)
    base = e.astype(jnp.float32) * CAP
    iota_c = jax.lax.broadcasted_iota(jnp.int32, (GS, CAP), 1).astype(jnp.float32)
    w1 = w1_ref[0]
    w2 = w2_ref[0]
    b1 = b1_ref[0]
    b2 = b2_ref[0]

    for g in range(G):
        cols = cols_ref[pl.ds(g * GS, GS), :]

        @pl.when(m == 0)
        def _():
            d0 = (cols[:, 0:1] - base) == iota_c
            d1 = (cols[:, 2:3] - base) == iota_c
            disp = (d0 | d1).astype(jnp.bfloat16)
            xe_ref[g] = jax.lax.dot_general(
                disp, x_ref[g], (((0,), (0,)), ((), ())),
                preferred_element_type=jnp.float32,
            ).astype(jnp.bfloat16)

        h = jax.lax.dot_general(
            xe_ref[g], w1, (((1,), (0,)), ((), ())),
            preferred_element_type=jnp.float32) + b1
        h = jax.nn.gelu(h).astype(jnp.bfloat16)
        t = jax.lax.dot_general(
            h, w2, (((1,), (0,)), ((), ())),
            preferred_element_type=jnp.float32)

        @pl.when(m == 0)
        def _():
            y_ref[g] = t + b2

        @pl.when(m == 1)
        def _():
            d0 = (cols[:, 0:1] - base) == iota_c
            d1 = (cols[:, 2:3] - base) == iota_c
            comb = (d0.astype(jnp.float32) * cols[:, 1:2]
                    + d1.astype(jnp.float32) * cols[:, 3:4])
            acc = jax.lax.dot_general(
                comb, y_ref[g] + t, (((1,), (0,)), ((), ())),
                preferred_element_type=jnp.float32)

            @pl.when(e == 0)
            def _():
                out_ref[g] = acc

            @pl.when(e > 0)
            def _():
                out_ref[g] += acc


def kernel(inputs, w_router, w1, b1, w2, b2):
    x = inputs.reshape(G, GS, D)
    rf = jnp.float32
    router_out = pl.pallas_call(
        _router_body,
        out_shape=[
            jax.ShapeDtypeStruct((G * GS, 8), rf),
            jax.ShapeDtypeStruct((1, 1), rf),
        ],
    )(x, w_router)
    cols, aux = router_out

    MC = 2
    MB = MLP // MC
    out = pl.pallas_call(
        _moe_body,
        grid=(E, MC),
        in_specs=[
            pl.BlockSpec((G, GS, D), lambda e, m: (0, 0, 0)),
            pl.BlockSpec((G * GS, 8), lambda e, m: (0, 0)),
            pl.BlockSpec((1, D, MB), lambda e, m: (e, 0, m)),
            pl.BlockSpec((1, 1, MB), lambda e, m: (e, 0, m)),
            pl.BlockSpec((1, MB, D), lambda e, m: (e, m, 0)),
            pl.BlockSpec((1, 1, D), lambda e, m: (e, 0, 0)),
        ],
        out_specs=pl.BlockSpec((G, GS, D), lambda e, m: (0, 0, 0)),
        out_shape=jax.ShapeDtypeStruct((G, GS, D), jnp.float32),
        scratch_shapes=[pltpu.VMEM((G, CAP, D), jnp.bfloat16),
                        pltpu.VMEM((G, CAP, D), jnp.float32)],
        compiler_params=pltpu.CompilerParams(
            dimension_semantics=("arbitrary", "arbitrary")),
    )(x.astype(jnp.bfloat16), cols, w1.astype(jnp.bfloat16),
      b1.reshape(E, 1, MLP), w2.astype(jnp.bfloat16), b2.reshape(E, 1, D))

    out = out.reshape(B, S, D)
    aux = aux.reshape(())
    return out, {"auxiliary_loss": aux, "importance_loss": aux}


# speedup vs baseline: 1.1247x; 1.1247x over previous
import jax
import jax.numpy as jnp
from jax.experimental import pallas as pl
from jax.experimental.pallas import tpu as pltpu

B, S, D = 2, 2048, 768
E, K = 8, 2
GS = 1024
MLP = 3072
CAP = (GS * K) // E
G = (B * S) // GS


def _router_body(x_ref, wr_ref, cols_ref, aux_ref):
    wr = wr_ref[...]
    iota_e = jax.lax.broadcasted_iota(jnp.int32, (GS, E), 1).astype(jnp.float32)
    tri =(jax.lax.broadcasted_iota(jnp.int32, (GS, GS), 1)
           < jax.lax.broadcasted_iota(jnp.int32, (GS, GS), 0)
           ).astype(jnp.float32)
    imps = []
    for g in range(G):
        xg = x_ref[g]
        logits = jax.lax.dot_general(
            xg, wr, (((1,), (0,)), ((), ())),
            preferred_element_type=jnp.float32)
        gates = jax.nn.softmax(logits, axis=-1)
        m1 = jnp.max(gates, axis=-1, keepdims=True)
        i1 = jnp.min(jnp.where(gates == m1, iota_e, float(E)), axis=-1,
                     keepdims=True)
        oh1 = (iota_e == i1).astype(jnp.float32)
        gmask = gates - oh1 * 1e30
        m2 = jnp.max(gmask, axis=-1, keepdims=True)
        i2 = jnp.min(jnp.where(gmask == m2, iota_e, float(E)), axis=-1,
                     keepdims=True)
        oh2 = (iota_e == i2).astype(jnp.float32)
        exc = jax.lax.dot_general(
            tri, jnp.concatenate([oh1, oh2], axis=1),
            (((1,), (0,)), ((), ())), preferred_element_type=jnp.float32)
        tot0 = jnp.sum(oh1, axis=0, keepdims=True)
        p0 = jnp.sum(oh1 * exc[:, :E], axis=-1, keepdims=True)
        p1 = jnp.sum(oh2 * (exc[:, E:] + tot0), axis=-1, keepdims=True)
        k0 = (p0 < CAP).astype(jnp.float32)
        k1 = (p1 < CAP).astype(jnp.float32)
        gate0 = jnp.sum(oh1 * gates, axis=-1, keepdims=True)
        gate1 = jnp.sum(oh2 * gates, axis=-1, keepdims=True)
        rows = pl.ds(g * GS, GS)
        cols_ref[rows, :] = jnp.concatenate([
            jnp.where(k0 > 0, i1 * CAP + p0, -1.0), gate0 * k0,
            jnp.where(k1 > 0, i2 * CAP + p1, -1.0), gate1 * k1,
            jnp.zeros((GS, 4), jnp.float32)], axis=1)
        imps.append(jnp.sum(gates, axis=0, keepdims=True))
    imp = jnp.concatenate(imps, axis=0)
    mean = jnp.mean(imp, axis=-1, keepdims=True)
    var = jnp.mean((imp - mean) ** 2, axis=-1, keepdims=True)
    aux = jnp.mean(var / (mean + 1e-10) ** 2)
    aux_ref[...] = jnp.full((1, 1), aux, jnp.float32)


def _moe_body(x_ref, cols_ref, w1_ref, b1_ref, w2_ref, b2_ref, out_ref):
    e = pl.program_id(0)
    base = e.astype(jnp.float32) * CAP
    iota_c = jax.lax.broadcasted_iota(jnp.int32, (GS, CAP), 1).astype(jnp.float32)
    w1 = w1_ref[0]
    w2 = w2_ref[0]
    b1 = b1_ref[0]
    b2 = b2_ref[0]

    for g in range(G):
        cols = cols_ref[pl.ds(g * GS, GS), :]
        d0 = (cols[:, 0:1] - base) == iota_c
        d1 = (cols[:, 2:3] - base) == iota_c
        comb = (d0.astype(jnp.float32) * cols[:, 1:2]
                + d1.astype(jnp.float32) * cols[:, 3:4])
        disp = (d0 | d1).astype(jnp.bfloat16)
        xe = jax.lax.dot_general(
            disp, x_ref[g], (((0,), (0,)), ((), ())),
            preferred_element_type=jnp.float32,
        ).astype(jnp.bfloat16)
        h = jax.lax.dot_general(
            xe, w1, (((1,), (0,)), ((), ())),
            preferred_element_type=jnp.float32) + b1
        h = jax.nn.gelu(h).astype(jnp.bfloat16)
        y = jax.lax.dot_general(
            h, w2, (((1,), (0,)), ((), ())),
            preferred_element_type=jnp.float32) + b2
        acc = jax.lax.dot_general(
            comb, y, (((1,), (0,)), ((), ())),
            preferred_element_type=jnp.float32)

        @pl.when(e == 0)
        def _():
            out_ref[g] = acc

        @pl.when(e > 0)
        def _():
            out_ref[g] += acc


def kernel(inputs, w_router, w1, b1, w2, b2):
    x = inputs.reshape(G, GS, D)
    rf = jnp.float32
    router_out = pl.pallas_call(
        _router_body,
        out_shape=[
            jax.ShapeDtypeStruct((G * GS, 8), rf),
            jax.ShapeDtypeStruct((1, 1), rf),
        ],
    )(x, w_router)
    cols, aux = router_out

    out = pl.pallas_call(
        _moe_body,
        grid=(E,),
        in_specs=[
            pl.BlockSpec((G, GS, D), lambda e: (0, 0, 0)),
            pl.BlockSpec((G * GS, 8), lambda e: (0, 0)),
            pl.BlockSpec((1, D, MLP), lambda e: (e, 0, 0)),
            pl.BlockSpec((1, 1, MLP), lambda e: (e, 0, 0)),
            pl.BlockSpec((1, MLP, D), lambda e: (e, 0, 0)),
            pl.BlockSpec((1, 1, D), lambda e: (e, 0, 0)),
        ],
        out_specs=pl.BlockSpec((G, GS, D), lambda e: (0, 0, 0)),
        out_shape=jax.ShapeDtypeStruct((G, GS, D), jnp.float32),
        compiler_params=pltpu.CompilerParams(
            dimension_semantics=("arbitrary",)),
    )(x.astype(jnp.bfloat16), cols, w1.astype(jnp.bfloat16),
      b1.reshape(E, 1, MLP), w2.astype(jnp.bfloat16), b2.reshape(E, 1, D))

    out = out.reshape(B, S, D)
    aux = aux.reshape(())
    return out, {"auxiliary_loss": aux, "importance_loss": aux}


# speedup vs baseline: 1.1747x; 1.0445x over previous
import functools

import jax
import jax.numpy as jnp
from jax import lax
from jax.experimental import pallas as pl
from jax.experimental.pallas import tpu as pltpu
from jax.experimental.pallas import tpu_sc as plsc

B, S, D = 2, 2048, 768
E, K = 8, 2
GS = 1024
MLP = 3072
CAP = (GS * K) // E
G = (B * S) // GS
TOK = B * S
NSLOT = E * G * CAP
NSLOT_PAD = NSLOT + 8
NC, NS = 2, 16
NW = NC * NS
TPW = TOK // NW


def _router_body(x_ref, wr_ref, cols_ref, ik0_ref, ik1_ref, aux_ref):
    wr = wr_ref[...]
    iota_e = jax.lax.broadcasted_iota(jnp.int32, (GS, E), 1).astype(jnp.float32)
    tri = (jax.lax.broadcasted_iota(jnp.int32, (GS, GS), 1)
           < jax.lax.broadcasted_iota(jnp.int32, (GS, GS), 0)
           ).astype(jnp.float32)
    imps = []
    for g in range(G):
        xg = x_ref[g]
        logits = jax.lax.dot_general(
            xg, wr, (((1,), (0,)), ((), ())),
            preferred_element_type=jnp.float32)
        gates = jax.nn.softmax(logits, axis=-1)
        m1 = jnp.max(gates, axis=-1, keepdims=True)
        i1 = jnp.min(jnp.where(gates == m1, iota_e, float(E)), axis=-1,
                     keepdims=True)
        oh1 = (iota_e == i1).astype(jnp.float32)
        gmask = gates - oh1 * 1e30
        m2 = jnp.max(gmask, axis=-1, keepdims=True)
        i2 = jnp.min(jnp.where(gmask == m2, iota_e, float(E)), axis=-1,
                     keepdims=True)
        oh2 = (iota_e == i2).astype(jnp.float32)
        exc = jax.lax.dot_general(
            tri, jnp.concatenate([oh1, oh2], axis=1),
            (((1,), (0,)), ((), ())), preferred_element_type=jnp.float32)
        tot0 = jnp.sum(oh1, axis=0, keepdims=True)
        p0 = jnp.sum(oh1 * exc[:, :E], axis=-1, keepdims=True)
        p1 = jnp.sum(oh2 * (exc[:, E:] + tot0), axis=-1, keepdims=True)
        k0 = (p0 < CAP).astype(jnp.float32)
        k1 = (p1 < CAP).astype(jnp.float32)
        gate0 = jnp.sum(oh1 * gates, axis=-1, keepdims=True)
        gate1 = jnp.sum(oh2 * gates, axis=-1, keepdims=True)
        rows = pl.ds(g * GS, GS)
        cols_ref[rows, :] = jnp.concatenate([
            jnp.where(k0 > 0, i1 * CAP + p0, -1.0), gate0 * k0,
            jnp.where(k1 > 0, i2 * CAP + p1, -1.0), gate1 * k1,
            jnp.zeros((GS, 4), jnp.float32)], axis=1)
        ik0_ref[rows, :] = jnp.where(
            k0 > 0, (i1 * G + g) * CAP + p0, float(NSLOT)).astype(jnp.int32)
        ik1_ref[rows, :] = jnp.where(
            k1 > 0, (i2 * G + g) * CAP + p1, float(NSLOT)).astype(jnp.int32)
        imps.append(jnp.sum(gates, axis=0, keepdims=True))
    imp = jnp.concatenate(imps, axis=0)
    mean = jnp.mean(imp, axis=-1, keepdims=True)
    var = jnp.mean((imp - mean) ** 2, axis=-1, keepdims=True)
    aux = jnp.mean(var / (mean + 1e-10) ** 2)
    aux_ref[...] = jnp.full((1, 1), aux, jnp.float32)


def _disp_body(ik0_hbm, ik1_hbm, x_hbm, xe_hbm, idx0_v, idx1_v, rows_v, sem):
    wid = lax.axis_index("s") * NC + lax.axis_index("c")
    base = wid * TPW
    pltpu.sync_copy(x_hbm.at[pl.ds(base, TPW)], rows_v)
    pltpu.sync_copy(ik0_hbm.at[pl.ds(base, TPW)], idx0_v)
    pltpu.sync_copy(ik1_hbm.at[pl.ds(base, TPW)], idx1_v)
    pltpu.async_copy(rows_v, xe_hbm.at[idx0_v], sem).wait()
    pltpu.async_copy(rows_v, xe_hbm.at[idx1_v], sem).wait()


def _moe_body(xe_ref, cols_ref, w1_ref, b1_ref, w2_ref, b2_ref, out_ref):
    e = pl.program_id(0)
    base = e.astype(jnp.float32) * CAP
    iota_c = jax.lax.broadcasted_iota(jnp.int32, (GS, CAP), 1).astype(jnp.float32)
    w1 = w1_ref[0]
    w2 = w2_ref[0]
    b1 = b1_ref[0]
    b2 = b2_ref[0]

    for g in range(G):
        cols = cols_ref[pl.ds(g * GS, GS), :]
        d0 = (cols[:, 0:1] - base) == iota_c
        d1 = (cols[:, 2:3] - base) == iota_c
        comb = (d0.astype(jnp.float32) * cols[:, 1:2]
                + d1.astype(jnp.float32) * cols[:, 3:4])
        xe = xe_ref[pl.ds(g * CAP, CAP), :].astype(jnp.bfloat16)
        h = jax.lax.dot_general(
            xe, w1, (((1,), (0,)), ((), ())),
            preferred_element_type=jnp.float32) + b1
        h = jax.nn.gelu(h).astype(jnp.bfloat16)
        y = jax.lax.dot_general(
            h, w2, (((1,), (0,)), ((), ())),
            preferred_element_type=jnp.float32) + b2
        acc = jax.lax.dot_general(
            comb, y, (((1,), (0,)), ((), ())),
            preferred_element_type=jnp.float32)

        @pl.when(e == 0)
        def _():
            out_ref[g] = acc

        @pl.when(e > 0)
        def _():
            out_ref[g] += acc


def kernel(inputs, w_router, w1, b1, w2, b2):
    x = inputs.reshape(G, GS, D)
    rf = jnp.float32
    cols, ik0, ik1, aux = pl.pallas_call(
        _router_body,
        out_shape=[
            jax.ShapeDtypeStruct((G * GS, 8), rf),
            jax.ShapeDtypeStruct((TOK, 1), jnp.int32),
            jax.ShapeDtypeStruct((TOK, 1), jnp.int32),
            jax.ShapeDtypeStruct((1, 1), rf),
        ],
    )(x, w_router)

    disp_call = pl.kernel(
        _disp_body,
        mesh=plsc.VectorSubcoreMesh(core_axis_name="c", subcore_axis_name="s"),
        out_type=jax.ShapeDtypeStruct((NSLOT_PAD, D), jnp.float32),
        scratch_types=[
            pltpu.VMEM((TPW,), jnp.int32),
            pltpu.VMEM((TPW,), jnp.int32),
            pltpu.VMEM((TPW, D), jnp.float32),
            pltpu.SemaphoreType.DMA,
        ],
    )
    xe = disp_call(ik0.reshape(TOK), ik1.reshape(TOK), inputs.reshape(TOK, D))

    out = pl.pallas_call(
        _moe_body,
        grid=(E,),
        in_specs=[
            pl.BlockSpec((G * CAP, D), lambda e: (e, 0)),
            pl.BlockSpec((G * GS, 8), lambda e: (0, 0)),
            pl.BlockSpec((1, D, MLP), lambda e: (e, 0, 0)),
            pl.BlockSpec((1, 1, MLP), lambda e: (e, 0, 0)),
            pl.BlockSpec((1, MLP, D), lambda e: (e, 0, 0)),
            pl.BlockSpec((1, 1, D), lambda e: (e, 0, 0)),
        ],
        out_specs=pl.BlockSpec((G, GS, D), lambda e: (0, 0, 0)),
        out_shape=jax.ShapeDtypeStruct((G, GS, D), jnp.float32),
        compiler_params=pltpu.CompilerParams(
            dimension_semantics=("arbitrary",)),
    )(xe, cols, w1.astype(jnp.bfloat16), b1.reshape(E, 1, MLP),
      w2.astype(jnp.bfloat16), b2.reshape(E, 1, D))

    out = out.reshape(B, S, D)
    aux = aux.reshape(())
    return out, {"auxiliary_loss": aux, "importance_loss": aux}
